# Initial kernel scaffold; baseline (speedup 1.0000x reference)
import jax
import jax.numpy as jnp
from jax.experimental import pallas as pl


def kernel(nodes, edges, senders, receivers, eW0, eb0, eW1, eb1, eg, ebeta, nW0, nb0, nW1, nb1, ng, nbeta):
    raise NotImplementedError("write your pallas kernel here")


# speedup vs baseline: 2.0322x; 2.0322x over previous
import functools

import jax
import jax.numpy as jnp
from jax import lax
from jax.experimental import pallas as pl
from jax.experimental.pallas import tpu as pltpu
from jax.experimental.pallas import tpu_sc as plsc

N = 10000
E = 320000
D_NODE = 128
D_EDGE = 16
HID = 128

NC = 2
NS = 16
NW = NC * NS
EPW = E // NW
CH = 80
NCHUNK = EPW // CH

_SC_MESH = plsc.VectorSubcoreMesh(core_axis_name="c", subcore_axis_name="s")


def _prep_body(x_ref, ws_ref, wr_ref, wn_ref, nb0_ref, ps_ref, pr_ref, pn_ref):
    x = x_ref[...]
    ps_ref[...] = jnp.dot(x, ws_ref[...], preferred_element_type=jnp.float32)
    pr_ref[...] = jnp.dot(x, wr_ref[...], preferred_element_type=jnp.float32)
    pn_ref[...] = (
        jnp.dot(x, wn_ref[...], preferred_element_type=jnp.float32) + nb0_ref[...]
    )


def _edge_body(hs_ref, hr_ref, e_ref, oh_ref, w0e_ref, b0_ref, w1_ref, b1_ref,
               g_ref, beta_ref, oe_ref, ue128_ref):
    e = e_ref[...]
    h = hs_ref[...] + hr_ref[...]
    h = h + jnp.dot(e, w0e_ref[...], preferred_element_type=jnp.float32)
    h = jnp.maximum(h + b0_ref[...], 0.0)
    up = jnp.dot(h, w1_ref[...], preferred_element_type=jnp.float32) + b1_ref[...]
    mu = jnp.mean(up, axis=-1, keepdims=True)
    d = up - mu
    var = jnp.mean(d * d, axis=-1, keepdims=True)
    ue = d * lax.rsqrt(var + 1e-5) * g_ref[...] + beta_ref[...]
    oe_ref[...] = e + ue
    oh = oh_ref[...]
    ue128_ref[...] = jnp.concatenate(
        [ue * oh[:, q:q + 1] for q in range(8)], axis=1)


def _node_body(pn_ref, agg_ref, x_ref, w0b_ref, w1_ref, b1_ref, g_ref, beta_ref,
               on_ref):
    a = agg_ref[0] + agg_ref[1]
    nf = jnp.maximum(
        pn_ref[...] + jnp.dot(a, w0b_ref[...], preferred_element_type=jnp.float32),
        0.0,
    )
    up = jnp.dot(nf, w1_ref[...], preferred_element_type=jnp.float32) + b1_ref[...]
    mu = jnp.mean(up, axis=-1, keepdims=True)
    d = up - mu
    var = jnp.mean(d * d, axis=-1, keepdims=True)
    un = d * lax.rsqrt(var + 1e-5) * g_ref[...] + beta_ref[...]
    on_ref[...] = x_ref[...] + un


@functools.partial(
    pl.kernel,
    out_type=(
        jax.ShapeDtypeStruct((E, D_NODE), jnp.float32),
        jax.ShapeDtypeStruct((E, D_NODE), jnp.float32),
    ),
    mesh=_SC_MESH,
    scratch_types=[
        pltpu.VMEM((NCHUNK, CH), jnp.int32),
        pltpu.VMEM((NCHUNK, CH), jnp.int32),
        pltpu.VMEM((CH, D_NODE), jnp.float32),
        pltpu.VMEM((CH, D_NODE), jnp.float32),
        pltpu.SemaphoreType.DMA,
        pltpu.SemaphoreType.DMA,
    ],
)
def _sc_gather(ps_hbm, pr_hbm, s_hbm, r_hbm, hs_hbm, hr_hbm,
               idx_s, idx_r, rows_s, rows_r, sem_s, sem_r):
    c = lax.axis_index("c")
    s = lax.axis_index("s")
    w = s * NC + c
    pltpu.sync_copy(s_hbm.at[w], idx_s)
    pltpu.sync_copy(r_hbm.at[w], idx_r)
    base = w * EPW

    def body(j, carry):
        cs = pltpu.async_copy(ps_hbm.at[idx_s.at[j]], rows_s, sem_s)
        cr = pltpu.async_copy(pr_hbm.at[idx_r.at[j]], rows_r, sem_r)
        cs.wait()
        cr.wait()
        pltpu.sync_copy(rows_s, hs_hbm.at[pl.ds(base + j * CH, CH), :])
        pltpu.sync_copy(rows_r, hr_hbm.at[pl.ds(base + j * CH, CH), :])
        return carry

    lax.fori_loop(0, NCHUNK, body, 0)


NG = N // 8


@functools.partial(
    pl.kernel,
    out_type=jax.ShapeDtypeStruct((NC, NG, 128), jnp.float32),
    mesh=_SC_MESH,
    scratch_types=[
        pltpu.VMEM((NCHUNK, CH), jnp.int32),
        pltpu.VMEM((CH, 128), jnp.float32),
        pltpu.VMEM_SHARED((NG, 128), jnp.float32),
    ],
)
def _sc_scatter(ue_hbm, g_hbm, z_hbm, out_hbm, idx_g, rows, acc):
    c = lax.axis_index("c")
    s = lax.axis_index("s")
    w = s * NC + c

    @pl.when(s == 0)
    def _():
        pltpu.sync_copy(z_hbm, acc)

    pltpu.sync_copy(g_hbm.at[w], idx_g)
    plsc.subcore_barrier()
    base = w * EPW

    def body(j, carry):
        pltpu.sync_copy(ue_hbm.at[pl.ds(base + j * CH, CH), :], rows)
        pltpu.sync_copy(rows, acc.at[idx_g.at[j]], add=True)
        return carry

    lax.fori_loop(0, NCHUNK, body, 0)
    plsc.subcore_barrier()

    @pl.when(s == 0)
    def _():
        pltpu.sync_copy(acc, out_hbm.at[c])


def kernel(nodes, edges, senders, receivers, eW0, eb0, eW1, eb1, eg, ebeta,
           nW0, nb0, nW1, nb1, ng, nbeta):
    f32 = jnp.float32
    eWe = eW0[:D_EDGE]
    eWs = eW0[D_EDGE:D_EDGE + D_NODE]
    eWr = eW0[D_EDGE + D_NODE:]
    nW0a = nW0[:D_NODE]
    nW0b = nW0[D_NODE:]
    eb0_2 = eb0.reshape(1, HID)
    eb1_2 = eb1.reshape(1, D_EDGE)
    eg_2 = eg.reshape(1, D_EDGE)
    ebeta_2 = ebeta.reshape(1, D_EDGE)
    nb0_2 = nb0.reshape(1, HID)
    nb1_2 = nb1.reshape(1, D_NODE)
    ng_2 = ng.reshape(1, D_NODE)
    nbeta_2 = nbeta.reshape(1, D_NODE)

    BN = 2000
    ps, pr, pn = pl.pallas_call(
        _prep_body,
        grid=(N // BN,),
        in_specs=[
            pl.BlockSpec((BN, D_NODE), lambda i: (i, 0)),
            pl.BlockSpec((D_NODE, HID), lambda i: (0, 0)),
            pl.BlockSpec((D_NODE, HID), lambda i: (0, 0)),
            pl.BlockSpec((D_NODE, HID), lambda i: (0, 0)),
            pl.BlockSpec((1, HID), lambda i: (0, 0)),
        ],
        out_specs=[
            pl.BlockSpec((BN, HID), lambda i: (i, 0)),
            pl.BlockSpec((BN, HID), lambda i: (i, 0)),
            pl.BlockSpec((BN, HID), lambda i: (i, 0)),
        ],
        out_shape=[
            jax.ShapeDtypeStruct((N, HID), f32),
            jax.ShapeDtypeStruct((N, HID), f32),
            jax.ShapeDtypeStruct((N, HID), f32),
        ],
    )(nodes, eWs, eWr, nW0a, nb0_2)

    s3 = senders.reshape(NW, NCHUNK, CH)
    r3 = receivers.reshape(NW, NCHUNK, CH)
    hs, hr = _sc_gather(ps, pr, s3, r3)

    onehot8 = (receivers[:, None] % 8 == jnp.arange(8)[None, :]).astype(f32)
    BE = 4000
    out_edges, ue128 = pl.pallas_call(
        _edge_body,
        grid=(E // BE,),
        in_specs=[
            pl.BlockSpec((BE, HID), lambda i: (i, 0)),
            pl.BlockSpec((BE, HID), lambda i: (i, 0)),
            pl.BlockSpec((BE, D_EDGE), lambda i: (i, 0)),
            pl.BlockSpec((BE, 8), lambda i: (i, 0)),
            pl.BlockSpec((D_EDGE, HID), lambda i: (0, 0)),
            pl.BlockSpec((1, HID), lambda i: (0, 0)),
            pl.BlockSpec((HID, D_EDGE), lambda i: (0, 0)),
            pl.BlockSpec((1, D_EDGE), lambda i: (0, 0)),
            pl.BlockSpec((1, D_EDGE), lambda i: (0, 0)),
            pl.BlockSpec((1, D_EDGE), lambda i: (0, 0)),
        ],
        out_specs=[
            pl.BlockSpec((BE, D_EDGE), lambda i: (i, 0)),
            pl.BlockSpec((BE, 128), lambda i: (i, 0)),
        ],
        out_shape=[
            jax.ShapeDtypeStruct((E, D_EDGE), f32),
            jax.ShapeDtypeStruct((E, 128), f32),
        ],
    )(hs, hr, edges, onehot8, eWe, eb0_2, eW1, eb1_2, eg_2, ebeta_2)

    g3 = (receivers // 8).reshape(NW, NCHUNK, CH)
    zeros = jnp.zeros((NG, 128), f32)
    agg2 = _sc_scatter(ue128, g3, zeros).reshape(NC, N, D_EDGE)

    out_nodes = pl.pallas_call(
        _node_body,
        grid=(N // BN,),
        in_specs=[
            pl.BlockSpec((BN, HID), lambda i: (i, 0)),
            pl.BlockSpec((NC, BN, D_EDGE), lambda i: (0, i, 0)),
            pl.BlockSpec((BN, D_NODE), lambda i: (i, 0)),
            pl.BlockSpec((D_EDGE, HID), lambda i: (0, 0)),
            pl.BlockSpec((HID, D_NODE), lambda i: (0, 0)),
            pl.BlockSpec((1, D_NODE), lambda i: (0, 0)),
            pl.BlockSpec((1, D_NODE), lambda i: (0, 0)),
            pl.BlockSpec((1, D_NODE), lambda i: (0, 0)),
        ],
        out_specs=pl.BlockSpec((BN, D_NODE), lambda i: (i, 0)),
        out_shape=jax.ShapeDtypeStruct((N, D_NODE), f32),
    )(pn, agg2, nodes, nW0b, nW1, nb1_2, ng_2, nbeta_2)

    return (out_nodes, out_edges)


# speedup vs baseline: 3.0396x; 1.4957x over previous
import functools

import jax
import jax.numpy as jnp
from jax import lax
from jax.experimental import pallas as pl
from jax.experimental.pallas import tpu as pltpu
from jax.experimental.pallas import tpu_sc as plsc

N = 10000
E = 320000
D_NODE = 128
D_EDGE = 16
HID = 128

NC = 2
NS = 16
NW = NC * NS
EPW = E // NW
CH = 80
NCHUNK = EPW // CH

_SC_MESH = plsc.VectorSubcoreMesh(core_axis_name="c", subcore_axis_name="s")


def _prep_body(x_ref, ws_ref, wr_ref, wn_ref, nb0_ref, ps_ref, pr_ref, pn_ref):
    x = x_ref[...]
    ps_ref[...] = jnp.dot(x, ws_ref[...], preferred_element_type=jnp.float32)
    pr_ref[...] = jnp.dot(x, wr_ref[...], preferred_element_type=jnp.float32)
    pn_ref[...] = (
        jnp.dot(x, wn_ref[...], preferred_element_type=jnp.float32) + nb0_ref[...]
    )


def _edge_body(h_ref, e_ref, r8_ref, w0e_ref, b0_ref, w1t_ref, b1t_ref,
               gt_ref, betat_ref, oe_ref, ue128_ref):
    e = e_ref[...]
    h = h_ref[...] + jnp.dot(e, w0e_ref[...], preferred_element_type=jnp.float32)
    h = jnp.maximum(h + b0_ref[...], 0.0)
    up = jnp.dot(h, w1t_ref[...], preferred_element_type=jnp.float32) + b1t_ref[...]
    mu = jnp.mean(up, axis=-1, keepdims=True)
    d = up - mu
    var = jnp.mean(d * d, axis=-1, keepdims=True)
    ue = d * lax.rsqrt(var + 1e-5) * gt_ref[...] + betat_ref[...]
    oe_ref[...] = e + ue[:, :D_EDGE]
    slot = lax.broadcasted_iota(jnp.int32, ue.shape, 1) // D_EDGE
    ue128_ref[...] = jnp.where(slot == r8_ref[...], ue, 0.0)


def _node_body(pn_ref, agg_ref, x_ref, w0b_ref, w1_ref, b1_ref, g_ref, beta_ref,
               on_ref):
    a = agg_ref[0] + agg_ref[1]
    nf = jnp.maximum(
        pn_ref[...] + jnp.dot(a, w0b_ref[...], preferred_element_type=jnp.float32),
        0.0,
    )
    up = jnp.dot(nf, w1_ref[...], preferred_element_type=jnp.float32) + b1_ref[...]
    mu = jnp.mean(up, axis=-1, keepdims=True)
    d = up - mu
    var = jnp.mean(d * d, axis=-1, keepdims=True)
    un = d * lax.rsqrt(var + 1e-5) * g_ref[...] + beta_ref[...]
    on_ref[...] = x_ref[...] + un


@functools.partial(
    pl.kernel,
    out_type=jax.ShapeDtypeStruct((E, D_NODE), jnp.float32),
    mesh=_SC_MESH,
    scratch_types=[
        pltpu.VMEM((NCHUNK, CH), jnp.int32),
        pltpu.VMEM((NCHUNK, CH), jnp.int32),
        pltpu.VMEM((CH, D_NODE), jnp.float32),
        pltpu.VMEM((CH, D_NODE), jnp.float32),
        pltpu.VMEM((CH,), jnp.int32),
        pltpu.VMEM_SHARED((NS * CH, D_NODE), jnp.float32),
        pltpu.SemaphoreType.DMA,
        pltpu.SemaphoreType.DMA,
    ],
)
def _sc_gather(ps_hbm, pr_hbm, s_hbm, r_hbm, iota_hbm, h_hbm,
               idx_s, idx_r, rows_s, rows_r, idx_dst, shared, sem_s, sem_r):
    c = lax.axis_index("c")
    s = lax.axis_index("s")
    w = s * NC + c
    pltpu.sync_copy(s_hbm.at[w], idx_s)
    pltpu.sync_copy(r_hbm.at[w], idx_r)
    pltpu.sync_copy(iota_hbm.at[s], idx_dst)
    base = w * EPW
    sl = shared.at[pl.ds(s * CH, CH), :]

    def body(j, carry):
        cs = pltpu.async_copy(ps_hbm.at[idx_s.at[j]], rows_s, sem_s)
        cr = pltpu.async_copy(pr_hbm.at[idx_r.at[j]], rows_r, sem_r)
        cs.wait()
        pltpu.sync_copy(rows_s, sl)
        cr.wait()
        pltpu.sync_copy(rows_r, shared.at[idx_dst], add=True)
        pltpu.sync_copy(sl, h_hbm.at[pl.ds(base + j * CH, CH), :])
        return carry

    lax.fori_loop(0, NCHUNK, body, 0)


NG = N // 8


@functools.partial(
    pl.kernel,
    out_type=jax.ShapeDtypeStruct((NC, NG, 128), jnp.float32),
    mesh=_SC_MESH,
    scratch_types=[
        pltpu.VMEM((NCHUNK, CH), jnp.int32),
        pltpu.VMEM((CH, 128), jnp.float32),
        pltpu.VMEM_SHARED((NG, 128), jnp.float32),
    ],
)
def _sc_scatter(ue_hbm, g_hbm, z_hbm, out_hbm, idx_g, rows, acc):
    c = lax.axis_index("c")
    s = lax.axis_index("s")
    w = s * NC + c

    @pl.when(s == 0)
    def _():
        pltpu.sync_copy(z_hbm, acc)

    pltpu.sync_copy(g_hbm.at[w], idx_g)
    plsc.subcore_barrier()
    base = w * EPW

    def body(j, carry):
        pltpu.sync_copy(ue_hbm.at[pl.ds(base + j * CH, CH), :], rows)
        pltpu.sync_copy(rows, acc.at[idx_g.at[j]], add=True)
        return carry

    lax.fori_loop(0, NCHUNK, body, 0)
    plsc.subcore_barrier()

    @pl.when(s == 0)
    def _():
        pltpu.sync_copy(acc, out_hbm.at[c])


def kernel(nodes, edges, senders, receivers, eW0, eb0, eW1, eb1, eg, ebeta,
           nW0, nb0, nW1, nb1, ng, nbeta):
    f32 = jnp.float32
    eWe = eW0[:D_EDGE]
    eWs = eW0[D_EDGE:D_EDGE + D_NODE]
    eWr = eW0[D_EDGE + D_NODE:]
    nW0a = nW0[:D_NODE]
    nW0b = nW0[D_NODE:]
    eb0_2 = eb0.reshape(1, HID)
    eW1t = jnp.tile(eW1, (1, 8))
    eb1t = jnp.tile(eb1, 8).reshape(1, HID)
    egt = jnp.tile(eg, 8).reshape(1, HID)
    ebetat = jnp.tile(ebeta, 8).reshape(1, HID)
    nb0_2 = nb0.reshape(1, HID)
    nb1_2 = nb1.reshape(1, D_NODE)
    ng_2 = ng.reshape(1, D_NODE)
    nbeta_2 = nbeta.reshape(1, D_NODE)

    BN = 2000
    ps, pr, pn = pl.pallas_call(
        _prep_body,
        grid=(N // BN,),
        in_specs=[
            pl.BlockSpec((BN, D_NODE), lambda i: (i, 0)),
            pl.BlockSpec((D_NODE, HID), lambda i: (0, 0)),
            pl.BlockSpec((D_NODE, HID), lambda i: (0, 0)),
            pl.BlockSpec((D_NODE, HID), lambda i: (0, 0)),
            pl.BlockSpec((1, HID), lambda i: (0, 0)),
        ],
        out_specs=[
            pl.BlockSpec((BN, HID), lambda i: (i, 0)),
            pl.BlockSpec((BN, HID), lambda i: (i, 0)),
            pl.BlockSpec((BN, HID), lambda i: (i, 0)),
        ],
        out_shape=[
            jax.ShapeDtypeStruct((N, HID), f32),
            jax.ShapeDtypeStruct((N, HID), f32),
            jax.ShapeDtypeStruct((N, HID), f32),
        ],
    )(nodes, eWs, eWr, nW0a, nb0_2)

    s3 = senders.reshape(NW, NCHUNK, CH)
    r3 = receivers.reshape(NW, NCHUNK, CH)
    iota2 = jnp.arange(NS * CH, dtype=jnp.int32).reshape(NS, CH)
    h = _sc_gather(ps, pr, s3, r3, iota2)

    r8 = (receivers % 8).reshape(E, 1)
    BE = 4000
    out_edges, ue128 = pl.pallas_call(
        _edge_body,
        grid=(E // BE,),
        in_specs=[
            pl.BlockSpec((BE, HID), lambda i: (i, 0)),
            pl.BlockSpec((BE, D_EDGE), lambda i: (i, 0)),
            pl.BlockSpec((BE, 1), lambda i: (i, 0)),
            pl.BlockSpec((D_EDGE, HID), lambda i: (0, 0)),
            pl.BlockSpec((1, HID), lambda i: (0, 0)),
            pl.BlockSpec((HID, HID), lambda i: (0, 0)),
            pl.BlockSpec((1, HID), lambda i: (0, 0)),
            pl.BlockSpec((1, HID), lambda i: (0, 0)),
            pl.BlockSpec((1, HID), lambda i: (0, 0)),
        ],
        out_specs=[
            pl.BlockSpec((BE, D_EDGE), lambda i: (i, 0)),
            pl.BlockSpec((BE, 128), lambda i: (i, 0)),
        ],
        out_shape=[
            jax.ShapeDtypeStruct((E, D_EDGE), f32),
            jax.ShapeDtypeStruct((E, 128), f32),
        ],
    )(h, edges, r8, eWe, eb0_2, eW1t, eb1t, egt, ebetat)

    g3 = (receivers // 8).reshape(NW, NCHUNK, CH)
    zeros = jnp.zeros((NG, 128), f32)
    agg2 = _sc_scatter(ue128, g3, zeros).reshape(NC, N, D_EDGE)

    out_nodes = pl.pallas_call(
        _node_body,
        grid=(N // BN,),
        in_specs=[
            pl.BlockSpec((BN, HID), lambda i: (i, 0)),
            pl.BlockSpec((NC, BN, D_EDGE), lambda i: (0, i, 0)),
            pl.BlockSpec((BN, D_NODE), lambda i: (i, 0)),
            pl.BlockSpec((D_EDGE, HID), lambda i: (0, 0)),
            pl.BlockSpec((HID, D_NODE), lambda i: (0, 0)),
            pl.BlockSpec((1, D_NODE), lambda i: (0, 0)),
            pl.BlockSpec((1, D_NODE), lambda i: (0, 0)),
            pl.BlockSpec((1, D_NODE), lambda i: (0, 0)),
        ],
        out_specs=pl.BlockSpec((BN, D_NODE), lambda i: (i, 0)),
        out_shape=jax.ShapeDtypeStruct((N, D_NODE), f32),
    )(pn, agg2, nodes, nW0b, nW1, nb1_2, ng_2, nbeta_2)

    return (out_nodes, out_edges)


# speedup vs baseline: 3.3100x; 1.0890x over previous
import functools

import jax
import jax.numpy as jnp
from jax import lax
from jax.experimental import pallas as pl
from jax.experimental.pallas import tpu as pltpu
from jax.experimental.pallas import tpu_sc as plsc

N = 10000
E = 320000
D_NODE = 128
D_EDGE = 16
HID = 128

NC = 2
NS = 16
NW = NC * NS
EPW = E // NW
CH = 80
NCHUNK = EPW // CH

_SC_MESH = plsc.VectorSubcoreMesh(core_axis_name="c", subcore_axis_name="s")


def _prep_body(x_ref, ws_ref, wr_ref, wn_ref, nb0_ref, ps_ref, pr_ref, pn_ref):
    x = x_ref[...]
    ps_ref[...] = jnp.dot(x, ws_ref[...], preferred_element_type=jnp.float32)
    pr_ref[...] = jnp.dot(x, wr_ref[...], preferred_element_type=jnp.float32)
    pn_ref[...] = (
        jnp.dot(x, wn_ref[...], preferred_element_type=jnp.float32) + nb0_ref[...]
    )


def _edge_body(h_ref, e_ref, r8_ref, w0e_ref, b0_ref, w1t_ref, b1t_ref,
               gt_ref, betat_ref, oe_ref, ue128_ref):
    e = e_ref[...]
    h = h_ref[...] + jnp.dot(e, w0e_ref[...], preferred_element_type=jnp.float32)
    h = jnp.maximum(h + b0_ref[...], 0.0)
    up = jnp.dot(h, w1t_ref[...], preferred_element_type=jnp.float32) + b1t_ref[...]
    mu = jnp.mean(up, axis=-1, keepdims=True)
    d = up - mu
    var = jnp.mean(d * d, axis=-1, keepdims=True)
    ue = d * lax.rsqrt(var + 1e-5) * gt_ref[...] + betat_ref[...]
    oe_ref[...] = e + ue[:, :D_EDGE]
    slot = lax.broadcasted_iota(jnp.int32, ue.shape, 1) // D_EDGE
    ue128_ref[...] = jnp.where(slot == r8_ref[...], ue, 0.0)


def _node_body(pn_ref, agg_ref, x_ref, w0b_ref, w1_ref, b1_ref, g_ref, beta_ref,
               on_ref):
    a = agg_ref[0] + agg_ref[1]
    nf = jnp.maximum(
        pn_ref[...] + jnp.dot(a, w0b_ref[...], preferred_element_type=jnp.float32),
        0.0,
    )
    up = jnp.dot(nf, w1_ref[...], preferred_element_type=jnp.float32) + b1_ref[...]
    mu = jnp.mean(up, axis=-1, keepdims=True)
    d = up - mu
    var = jnp.mean(d * d, axis=-1, keepdims=True)
    un = d * lax.rsqrt(var + 1e-5) * g_ref[...] + beta_ref[...]
    on_ref[...] = x_ref[...] + un


@functools.partial(
    pl.kernel,
    out_type=jax.ShapeDtypeStruct((E, D_NODE), jnp.float32),
    mesh=_SC_MESH,
    scratch_types=[
        pltpu.VMEM((NCHUNK, CH), jnp.int32),
        pltpu.VMEM((NCHUNK, CH), jnp.int32),
        pltpu.VMEM((2, CH, D_NODE), jnp.float32),
        pltpu.VMEM((2, CH, D_NODE), jnp.float32),
        pltpu.VMEM((2, CH), jnp.int32),
        pltpu.VMEM_SHARED((NS * 2 * CH, D_NODE), jnp.float32),
        pltpu.SemaphoreType.DMA,
        pltpu.SemaphoreType.DMA,
        pltpu.SemaphoreType.DMA,
        pltpu.SemaphoreType.DMA,
        pltpu.SemaphoreType.DMA,
        pltpu.SemaphoreType.DMA,
    ],
)
def _sc_gather(ps_hbm, pr_hbm, s_hbm, r_hbm, iota_hbm, h_hbm,
               idx_s, idx_r, rows_s, rows_r, idx_dst, shared,
               sem_s0, sem_s1, sem_r0, sem_r1, sem_o0, sem_o1):
    c = lax.axis_index("c")
    s = lax.axis_index("s")
    w = s * NC + c
    pltpu.sync_copy(s_hbm.at[w], idx_s)
    pltpu.sync_copy(r_hbm.at[w], idx_r)
    pltpu.sync_copy(iota_hbm.at[s], idx_dst)
    base = w * EPW
    sems_s = (sem_s0, sem_s1)
    sems_r = (sem_r0, sem_r1)
    sems_o = (sem_o0, sem_o1)

    def sl(b):
        return shared.at[pl.ds((s * 2 + b) * CH, CH), :]

    def start(j, b):
        pltpu.async_copy(ps_hbm.at[idx_s.at[j]], rows_s.at[b], sems_s[b])
        pltpu.async_copy(pr_hbm.at[idx_r.at[j]], rows_r.at[b], sems_r[b])

    start(0, 0)

    def outer(i, carry):
        j = i * 2
        for b in range(2):
            cur = j + b

            @pl.when(cur < NCHUNK)
            def _():
                @pl.when(cur + 1 < NCHUNK)
                def _():
                    start(cur + 1, 1 - b)

                pltpu.make_async_copy(
                    ps_hbm.at[idx_s.at[cur]], rows_s.at[b], sems_s[b]).wait()
                pltpu.make_async_copy(
                    pr_hbm.at[idx_r.at[cur]], rows_r.at[b], sems_r[b]).wait()

                @pl.when(cur >= 2)
                def _():
                    pltpu.make_async_copy(
                        sl(b),
                        h_hbm.at[pl.ds(base + (cur - 2) * CH, CH), :],
                        sems_o[b]).wait()

                pltpu.sync_copy(rows_s.at[b], sl(b))
                pltpu.sync_copy(rows_r.at[b], shared.at[idx_dst.at[b]],
                                add=True)
                pltpu.async_copy(sl(b),
                                 h_hbm.at[pl.ds(base + cur * CH, CH), :],
                                 sems_o[b])
        return carry

    lax.fori_loop(0, (NCHUNK + 1) // 2, outer, 0)
    for last in (NCHUNK - 2, NCHUNK - 1):
        b = last % 2
        pltpu.make_async_copy(
            sl(b), h_hbm.at[pl.ds(base + last * CH, CH), :], sems_o[b]).wait()


NG = N // 8


@functools.partial(
    pl.kernel,
    out_type=jax.ShapeDtypeStruct((NC, NG, 128), jnp.float32),
    mesh=_SC_MESH,
    scratch_types=[
        pltpu.VMEM((NCHUNK, CH), jnp.int32),
        pltpu.VMEM((CH, 128), jnp.float32),
        pltpu.VMEM_SHARED((NG, 128), jnp.float32),
    ],
)
def _sc_scatter(ue_hbm, g_hbm, z_hbm, out_hbm, idx_g, rows, acc):
    c = lax.axis_index("c")
    s = lax.axis_index("s")
    w = s * NC + c

    @pl.when(s == 0)
    def _():
        pltpu.sync_copy(z_hbm, acc)

    pltpu.sync_copy(g_hbm.at[w], idx_g)
    plsc.subcore_barrier()
    base = w * EPW

    def body(j, carry):
        pltpu.sync_copy(ue_hbm.at[pl.ds(base + j * CH, CH), :], rows)
        pltpu.sync_copy(rows, acc.at[idx_g.at[j]], add=True)
        return carry

    lax.fori_loop(0, NCHUNK, body, 0)
    plsc.subcore_barrier()

    @pl.when(s == 0)
    def _():
        pltpu.sync_copy(acc, out_hbm.at[c])


def kernel(nodes, edges, senders, receivers, eW0, eb0, eW1, eb1, eg, ebeta,
           nW0, nb0, nW1, nb1, ng, nbeta):
    f32 = jnp.float32
    eWe = eW0[:D_EDGE]
    eWs = eW0[D_EDGE:D_EDGE + D_NODE]
    eWr = eW0[D_EDGE + D_NODE:]
    nW0a = nW0[:D_NODE]
    nW0b = nW0[D_NODE:]
    eb0_2 = eb0.reshape(1, HID)
    eW1t = jnp.tile(eW1, (1, 8))
    eb1t = jnp.tile(eb1, 8).reshape(1, HID)
    egt = jnp.tile(eg, 8).reshape(1, HID)
    ebetat = jnp.tile(ebeta, 8).reshape(1, HID)
    nb0_2 = nb0.reshape(1, HID)
    nb1_2 = nb1.reshape(1, D_NODE)
    ng_2 = ng.reshape(1, D_NODE)
    nbeta_2 = nbeta.reshape(1, D_NODE)

    BN = 2000
    ps, pr, pn = pl.pallas_call(
        _prep_body,
        grid=(N // BN,),
        in_specs=[
            pl.BlockSpec((BN, D_NODE), lambda i: (i, 0)),
            pl.BlockSpec((D_NODE, HID), lambda i: (0, 0)),
            pl.BlockSpec((D_NODE, HID), lambda i: (0, 0)),
            pl.BlockSpec((D_NODE, HID), lambda i: (0, 0)),
            pl.BlockSpec((1, HID), lambda i: (0, 0)),
        ],
        out_specs=[
            pl.BlockSpec((BN, HID), lambda i: (i, 0)),
            pl.BlockSpec((BN, HID), lambda i: (i, 0)),
            pl.BlockSpec((BN, HID), lambda i: (i, 0)),
        ],
        out_shape=[
            jax.ShapeDtypeStruct((N, HID), f32),
            jax.ShapeDtypeStruct((N, HID), f32),
            jax.ShapeDtypeStruct((N, HID), f32),
        ],
    )(nodes, eWs, eWr, nW0a, nb0_2)

    s3 = senders.reshape(NW, NCHUNK, CH)
    r3 = receivers.reshape(NW, NCHUNK, CH)
    iota2 = jnp.arange(NS * 2 * CH, dtype=jnp.int32).reshape(NS, 2, CH)
    h = _sc_gather(ps, pr, s3, r3, iota2)

    r8 = (receivers % 8).reshape(E, 1)
    BE = 4000
    out_edges, ue128 = pl.pallas_call(
        _edge_body,
        grid=(E // BE,),
        in_specs=[
            pl.BlockSpec((BE, HID), lambda i: (i, 0)),
            pl.BlockSpec((BE, D_EDGE), lambda i: (i, 0)),
            pl.BlockSpec((BE, 1), lambda i: (i, 0)),
            pl.BlockSpec((D_EDGE, HID), lambda i: (0, 0)),
            pl.BlockSpec((1, HID), lambda i: (0, 0)),
            pl.BlockSpec((HID, HID), lambda i: (0, 0)),
            pl.BlockSpec((1, HID), lambda i: (0, 0)),
            pl.BlockSpec((1, HID), lambda i: (0, 0)),
            pl.BlockSpec((1, HID), lambda i: (0, 0)),
        ],
        out_specs=[
            pl.BlockSpec((BE, D_EDGE), lambda i: (i, 0)),
            pl.BlockSpec((BE, 128), lambda i: (i, 0)),
        ],
        out_shape=[
            jax.ShapeDtypeStruct((E, D_EDGE), f32),
            jax.ShapeDtypeStruct((E, 128), f32),
        ],
    )(h, edges, r8, eWe, eb0_2, eW1t, eb1t, egt, ebetat)

    g3 = (receivers // 8).reshape(NW, NCHUNK, CH)
    zeros = jnp.zeros((NG, 128), f32)
    agg2 = _sc_scatter(ue128, g3, zeros).reshape(NC, N, D_EDGE)

    out_nodes = pl.pallas_call(
        _node_body,
        grid=(N // BN,),
        in_specs=[
            pl.BlockSpec((BN, HID), lambda i: (i, 0)),
            pl.BlockSpec((NC, BN, D_EDGE), lambda i: (0, i, 0)),
            pl.BlockSpec((BN, D_NODE), lambda i: (i, 0)),
            pl.BlockSpec((D_EDGE, HID), lambda i: (0, 0)),
            pl.BlockSpec((HID, D_NODE), lambda i: (0, 0)),
            pl.BlockSpec((1, D_NODE), lambda i: (0, 0)),
            pl.BlockSpec((1, D_NODE), lambda i: (0, 0)),
            pl.BlockSpec((1, D_NODE), lambda i: (0, 0)),
        ],
        out_specs=pl.BlockSpec((BN, D_NODE), lambda i: (i, 0)),
        out_shape=jax.ShapeDtypeStruct((N, D_NODE), f32),
    )(pn, agg2, nodes, nW0b, nW1, nb1_2, ng_2, nbeta_2)

    return (out_nodes, out_edges)


# speedup vs baseline: 3.6427x; 1.1005x over previous
import functools

import jax
import jax.numpy as jnp
from jax import lax
from jax.experimental import pallas as pl
from jax.experimental.pallas import tpu as pltpu
from jax.experimental.pallas import tpu_sc as plsc

N = 10000
E = 320000
D_NODE = 128
D_EDGE = 16
HID = 128

NC = 2
NS = 16
NW = NC * NS
EPW = E // NW
CH = 80
NCHUNK = EPW // CH

_SC_MESH = plsc.VectorSubcoreMesh(core_axis_name="c", subcore_axis_name="s")


def _prep_body(x_ref, ws_ref, wr_ref, wn_ref, nb0_ref, ps_ref, pr_ref, pn_ref):
    x = x_ref[...]
    ps_ref[...] = jnp.dot(x, ws_ref[...], preferred_element_type=jnp.float32)
    pr_ref[...] = jnp.dot(x, wr_ref[...], preferred_element_type=jnp.float32)
    pn_ref[...] = (
        jnp.dot(x, wn_ref[...], preferred_element_type=jnp.float32) + nb0_ref[...]
    )


def _edge_body(h_ref, e_ref, r8_ref, w0e_ref, b0_ref, w1t_ref, b1t_ref,
               gt_ref, betat_ref, oe_ref, ue128_ref):
    e = e_ref[...]
    h = h_ref[...] + jnp.dot(e, w0e_ref[...], preferred_element_type=jnp.float32)
    h = jnp.maximum(h + b0_ref[...], 0.0)
    up = jnp.dot(h, w1t_ref[...], preferred_element_type=jnp.float32) + b1t_ref[...]
    mu = jnp.mean(up, axis=-1, keepdims=True)
    d = up - mu
    var = jnp.mean(d * d, axis=-1, keepdims=True)
    ue = d * lax.rsqrt(var + 1e-5) * gt_ref[...] + betat_ref[...]
    oe_ref[...] = e + ue[:, :D_EDGE]
    slot = lax.broadcasted_iota(jnp.int32, ue.shape, 1) // D_EDGE
    ue128_ref[...] = jnp.where(slot == r8_ref[...], ue, 0.0)


def _node_body(pn_ref, agg_ref, x_ref, w0b_ref, w1_ref, b1_ref, g_ref, beta_ref,
               on_ref):
    a = agg_ref[0] + agg_ref[1]
    nf = jnp.maximum(
        pn_ref[...] + jnp.dot(a, w0b_ref[...], preferred_element_type=jnp.float32),
        0.0,
    )
    up = jnp.dot(nf, w1_ref[...], preferred_element_type=jnp.float32) + b1_ref[...]
    mu = jnp.mean(up, axis=-1, keepdims=True)
    d = up - mu
    var = jnp.mean(d * d, axis=-1, keepdims=True)
    un = d * lax.rsqrt(var + 1e-5) * g_ref[...] + beta_ref[...]
    on_ref[...] = x_ref[...] + un


@functools.partial(
    pl.kernel,
    out_type=jax.ShapeDtypeStruct((E, D_NODE), jnp.float32),
    mesh=_SC_MESH,
    scratch_types=[
        pltpu.VMEM((NCHUNK, CH), jnp.int32),
        pltpu.VMEM((NCHUNK, CH), jnp.int32),
        pltpu.VMEM((2, CH, D_NODE), jnp.float32),
        pltpu.VMEM((2, CH, D_NODE), jnp.float32),
        pltpu.VMEM((2, CH), jnp.int32),
        pltpu.VMEM_SHARED((NS * 2 * CH, D_NODE), jnp.float32),
        pltpu.SemaphoreType.DMA,
        pltpu.SemaphoreType.DMA,
        pltpu.SemaphoreType.DMA,
        pltpu.SemaphoreType.DMA,
        pltpu.SemaphoreType.DMA,
        pltpu.SemaphoreType.DMA,
    ],
)
def _sc_gather(ps_hbm, pr_hbm, s_hbm, r_hbm, iota_hbm, h_hbm,
               idx_s, idx_r, rows_s, rows_r, idx_dst, shared,
               sem_s0, sem_s1, sem_r0, sem_r1, sem_o0, sem_o1):
    c = lax.axis_index("c")
    s = lax.axis_index("s")
    w = s * NC + c
    pltpu.sync_copy(s_hbm.at[w], idx_s)
    pltpu.sync_copy(r_hbm.at[w], idx_r)
    pltpu.sync_copy(iota_hbm.at[s], idx_dst)
    base = w * EPW
    sems_s = (sem_s0, sem_s1)
    sems_r = (sem_r0, sem_r1)
    sems_o = (sem_o0, sem_o1)

    def sl(b):
        return shared.at[pl.ds((s * 2 + b) * CH, CH), :]

    def start(j, b):
        pltpu.async_copy(ps_hbm.at[idx_s.at[j]], rows_s.at[b], sems_s[b])
        pltpu.async_copy(pr_hbm.at[idx_r.at[j]], rows_r.at[b], sems_r[b])

    start(0, 0)

    def outer(i, carry):
        j = i * 2
        for b in range(2):
            cur = j + b

            @pl.when(cur < NCHUNK)
            def _():
                @pl.when(cur + 1 < NCHUNK)
                def _():
                    start(cur + 1, 1 - b)

                pltpu.make_async_copy(
                    ps_hbm.at[idx_s.at[cur]], rows_s.at[b], sems_s[b]).wait()
                pltpu.make_async_copy(
                    pr_hbm.at[idx_r.at[cur]], rows_r.at[b], sems_r[b]).wait()

                @pl.when(cur >= 2)
                def _():
                    pltpu.make_async_copy(
                        sl(b),
                        h_hbm.at[pl.ds(base + (cur - 2) * CH, CH), :],
                        sems_o[b]).wait()

                pltpu.sync_copy(rows_s.at[b], sl(b))
                pltpu.sync_copy(rows_r.at[b], shared.at[idx_dst.at[b]],
                                add=True)
                pltpu.async_copy(sl(b),
                                 h_hbm.at[pl.ds(base + cur * CH, CH), :],
                                 sems_o[b])
        return carry

    lax.fori_loop(0, (NCHUNK + 1) // 2, outer, 0)
    for last in (NCHUNK - 2, NCHUNK - 1):
        b = last % 2
        pltpu.make_async_copy(
            sl(b), h_hbm.at[pl.ds(base + last * CH, CH), :], sems_o[b]).wait()


NG = N // 8


@functools.partial(
    pl.kernel,
    out_type=jax.ShapeDtypeStruct((NC, NG, 128), jnp.float32),
    mesh=_SC_MESH,
    scratch_types=[
        pltpu.VMEM((NCHUNK, CH), jnp.int32),
        pltpu.VMEM((2, CH, 128), jnp.float32),
        pltpu.VMEM_SHARED((NG, 128), jnp.float32),
        pltpu.SemaphoreType.DMA,
        pltpu.SemaphoreType.DMA,
    ],
)
def _sc_scatter(ue_hbm, g_hbm, z_hbm, out_hbm, idx_g, rows, acc, sem_i0, sem_i1):
    c = lax.axis_index("c")
    s = lax.axis_index("s")
    w = s * NC + c

    @pl.when(s == 0)
    def _():
        pltpu.sync_copy(z_hbm, acc)

    pltpu.sync_copy(g_hbm.at[w], idx_g)
    plsc.subcore_barrier()
    base = w * EPW
    sems_i = (sem_i0, sem_i1)

    def start(j, b):
        pltpu.async_copy(ue_hbm.at[pl.ds(base + j * CH, CH), :], rows.at[b],
                         sems_i[b])

    start(0, 0)

    def outer(i, carry):
        j = i * 2
        for b in range(2):
            cur = j + b

            @pl.when(cur < NCHUNK)
            def _():
                @pl.when(cur + 1 < NCHUNK)
                def _():
                    start(cur + 1, 1 - b)

                pltpu.make_async_copy(
                    ue_hbm.at[pl.ds(base + cur * CH, CH), :], rows.at[b],
                    sems_i[b]).wait()
                pltpu.sync_copy(rows.at[b], acc.at[idx_g.at[cur]], add=True)
        return carry

    lax.fori_loop(0, (NCHUNK + 1) // 2, outer, 0)
    plsc.subcore_barrier()

    @pl.when(s == 0)
    def _():
        pltpu.sync_copy(acc, out_hbm.at[c])


def kernel(nodes, edges, senders, receivers, eW0, eb0, eW1, eb1, eg, ebeta,
           nW0, nb0, nW1, nb1, ng, nbeta):
    f32 = jnp.float32
    eWe = eW0[:D_EDGE]
    eWs = eW0[D_EDGE:D_EDGE + D_NODE]
    eWr = eW0[D_EDGE + D_NODE:]
    nW0a = nW0[:D_NODE]
    nW0b = nW0[D_NODE:]
    eb0_2 = eb0.reshape(1, HID)
    eW1t = jnp.tile(eW1, (1, 8))
    eb1t = jnp.tile(eb1, 8).reshape(1, HID)
    egt = jnp.tile(eg, 8).reshape(1, HID)
    ebetat = jnp.tile(ebeta, 8).reshape(1, HID)
    nb0_2 = nb0.reshape(1, HID)
    nb1_2 = nb1.reshape(1, D_NODE)
    ng_2 = ng.reshape(1, D_NODE)
    nbeta_2 = nbeta.reshape(1, D_NODE)

    BN = 2000
    ps, pr, pn = pl.pallas_call(
        _prep_body,
        grid=(N // BN,),
        in_specs=[
            pl.BlockSpec((BN, D_NODE), lambda i: (i, 0)),
            pl.BlockSpec((D_NODE, HID), lambda i: (0, 0)),
            pl.BlockSpec((D_NODE, HID), lambda i: (0, 0)),
            pl.BlockSpec((D_NODE, HID), lambda i: (0, 0)),
            pl.BlockSpec((1, HID), lambda i: (0, 0)),
        ],
        out_specs=[
            pl.BlockSpec((BN, HID), lambda i: (i, 0)),
            pl.BlockSpec((BN, HID), lambda i: (i, 0)),
            pl.BlockSpec((BN, HID), lambda i: (i, 0)),
        ],
        out_shape=[
            jax.ShapeDtypeStruct((N, HID), f32),
            jax.ShapeDtypeStruct((N, HID), f32),
            jax.ShapeDtypeStruct((N, HID), f32),
        ],
    )(nodes, eWs, eWr, nW0a, nb0_2)

    s3 = senders.reshape(NW, NCHUNK, CH)
    r3 = receivers.reshape(NW, NCHUNK, CH)
    iota2 = jnp.arange(NS * 2 * CH, dtype=jnp.int32).reshape(NS, 2, CH)
    h = _sc_gather(ps, pr, s3, r3, iota2)

    r8 = (receivers % 8).reshape(E, 1)
    BE = 4000
    out_edges, ue128 = pl.pallas_call(
        _edge_body,
        grid=(E // BE,),
        in_specs=[
            pl.BlockSpec((BE, HID), lambda i: (i, 0)),
            pl.BlockSpec((BE, D_EDGE), lambda i: (i, 0)),
            pl.BlockSpec((BE, 1), lambda i: (i, 0)),
            pl.BlockSpec((D_EDGE, HID), lambda i: (0, 0)),
            pl.BlockSpec((1, HID), lambda i: (0, 0)),
            pl.BlockSpec((HID, HID), lambda i: (0, 0)),
            pl.BlockSpec((1, HID), lambda i: (0, 0)),
            pl.BlockSpec((1, HID), lambda i: (0, 0)),
            pl.BlockSpec((1, HID), lambda i: (0, 0)),
        ],
        out_specs=[
            pl.BlockSpec((BE, D_EDGE), lambda i: (i, 0)),
            pl.BlockSpec((BE, 128), lambda i: (i, 0)),
        ],
        out_shape=[
            jax.ShapeDtypeStruct((E, D_EDGE), f32),
            jax.ShapeDtypeStruct((E, 128), f32),
        ],
    )(h, edges, r8, eWe, eb0_2, eW1t, eb1t, egt, ebetat)

    g3 = (receivers // 8).reshape(NW, NCHUNK, CH)
    zeros = jnp.zeros((NG, 128), f32)
    agg2 = _sc_scatter(ue128, g3, zeros).reshape(NC, N, D_EDGE)

    out_nodes = pl.pallas_call(
        _node_body,
        grid=(N // BN,),
        in_specs=[
            pl.BlockSpec((BN, HID), lambda i: (i, 0)),
            pl.BlockSpec((NC, BN, D_EDGE), lambda i: (0, i, 0)),
            pl.BlockSpec((BN, D_NODE), lambda i: (i, 0)),
            pl.BlockSpec((D_EDGE, HID), lambda i: (0, 0)),
            pl.BlockSpec((HID, D_NODE), lambda i: (0, 0)),
            pl.BlockSpec((1, D_NODE), lambda i: (0, 0)),
            pl.BlockSpec((1, D_NODE), lambda i: (0, 0)),
            pl.BlockSpec((1, D_NODE), lambda i: (0, 0)),
        ],
        out_specs=pl.BlockSpec((BN, D_NODE), lambda i: (i, 0)),
        out_shape=jax.ShapeDtypeStruct((N, D_NODE), f32),
    )(pn, agg2, nodes, nW0b, nW1, nb1_2, ng_2, nbeta_2)

    return (out_nodes, out_edges)
